# baseline (speedup 1.0000x reference)
import functools

import jax
from jax import lax
import jax.numpy as jnp
from jax.experimental import pallas as pl
from jax.experimental.pallas import tpu as pltpu
from jax.experimental.pallas import tpu_sc as plsc

N, D, E, H, TOPK = 2048, 768, 8, 2048, 2
AUX_COEF = 0.01
TN = 128
NTILES = 40
NP = NTILES * TN

NC, NS = 2, 16
NW = NC * NS
AB = (2 * N) // NW
TB = N // NW


def _router_kernel(x_ref, gate_ref, pos0_ref, pos1_ref, p0_ref, p1_ref,
                   rid_ref, rexp_ref, aux_ref):
    lt = jax.lax.dot_general(gate_ref[...], x_ref[...],
                             (((1,), (1,)), ((), ())),
                             preferred_element_type=jnp.float32)
    m = jnp.max(lt, axis=0, keepdims=True)
    ex = jnp.exp(lt - m)
    p = ex / jnp.sum(ex, axis=0, keepdims=True)

    tpe = jnp.mean(p, axis=1, keepdims=True)
    aux = AUX_COEF * jnp.mean((tpe - 1.0 / E) ** 2)
    aux_ref[...] = aux.reshape(1, 1)

    ie = jax.lax.broadcasted_iota(jnp.int32, p.shape, 0)
    m1 = jnp.max(p, axis=0, keepdims=True)
    i1 = jnp.min(jnp.where(p == m1, ie, E), axis=0, keepdims=True)
    f1 = ie == i1
    pw = jnp.where(f1, -jnp.inf, p)
    m2 = jnp.max(pw, axis=0, keepdims=True)
    i2 = jnp.min(jnp.where(pw == m2, ie, E), axis=0, keepdims=True)
    f2 = ie == i2
    denom = m1 + m2
    p0_ref[...] = m1 / denom
    p1_ref[...] = m2 / denom

    oh0 = f1.astype(jnp.float32)
    oh1 = f2.astype(jnp.float32)

    it_r = jax.lax.broadcasted_iota(jnp.int32, (N, N), 0)
    it_c = jax.lax.broadcasted_iota(jnp.int32, (N, N), 1)
    tri = (it_r < it_c).astype(jnp.float32)
    rank0t = jax.lax.dot_general(oh0, tri, (((1,), (0,)), ((), ())),
                                 preferred_element_type=jnp.float32)
    rank1t = jax.lax.dot_general(oh1, tri, (((1,), (0,)), ((), ())),
                                 preferred_element_type=jnp.float32)
    rank0 = jnp.sum(rank0t * oh0, axis=0, keepdims=True)
    rank1 = jnp.sum(rank1t * oh1, axis=0, keepdims=True)

    c0 = jnp.sum(oh0, axis=1, keepdims=True)
    c1 = jnp.sum(oh1, axis=1, keepdims=True)
    ci = (c0 + c1).astype(jnp.int32)
    pc = jnp.right_shift(ci + (TN - 1), 7) << 7
    ie8r = jax.lax.broadcasted_iota(jnp.int32, (E, E), 0)
    ie8c = jax.lax.broadcasted_iota(jnp.int32, (E, E), 1)
    tri8 = (ie8r > ie8c).astype(jnp.float32)
    po = jax.lax.dot_general(tri8, pc.astype(jnp.float32),
                             (((1,), (0,)), ((), ())),
                             preferred_element_type=jnp.float32)

    pos0 = jnp.sum(po * oh0, axis=0, keepdims=True) + rank0
    pos1 = jnp.sum((po + c0) * oh1, axis=0, keepdims=True) + rank1
    pos0_ref[...] = pos0.astype(jnp.int32)
    pos1_ref[...] = pos1.astype(jnp.int32)

    nz = (pc > 0).astype(jnp.float32)
    itile = (jax.lax.broadcasted_iota(jnp.int32, (1, NTILES), 1)
             .astype(jnp.float32) * float(TN))
    started = nz * (po <= itile).astype(jnp.float32)
    rid_ref[...] = jnp.sum(started, axis=0, keepdims=True).astype(jnp.int32) - 1
    tri8i = (ie8r >= ie8c).astype(jnp.float32)
    nzcum = jax.lax.dot_general(tri8i, nz, (((1,), (0,)), ((), ())),
                                preferred_element_type=jnp.float32)
    rank_e = nzcum - 1.0
    ir = (jax.lax.broadcasted_iota(jnp.int32, (E, E), 1)
          .astype(jnp.float32))
    ev = (jax.lax.broadcasted_iota(jnp.int32, (E, E), 0)
          .astype(jnp.float32))
    msk = nz * (rank_e == ir).astype(jnp.float32)
    rexp = jnp.sum((ev + 1.0) * msk, axis=0, keepdims=True) - 1.0
    rexp_ref[...] = rexp.astype(jnp.int32)


def _group_kernel(rid_ref, rexp_ref, xg_ref, w1_ref, w2_ref, w3_ref, yg_ref,
                  b1, b2, b3, sems):
    i = pl.program_id(0)
    r = rid_ref[0, i]
    slot = lax.rem(r, 2)

    def issue(rr):
        sl = lax.rem(rr, 2)
        e = rexp_ref[0, jnp.minimum(rr, E - 1)]

        @pl.when(jnp.logical_and(rr <= E - 1, e >= 0))
        def _():
            pltpu.make_async_copy(w1_ref.at[e], b1.at[sl], sems.at[sl]).start()
            pltpu.make_async_copy(w2_ref.at[e], b2.at[sl], sems.at[sl]).start()
            pltpu.make_async_copy(w3_ref.at[e], b3.at[sl], sems.at[sl]).start()

    @pl.when(i == 0)
    def _prime():
        issue(jnp.int32(0))
        issue(jnp.int32(1))

    prev = rid_ref[0, jnp.maximum(i - 1, 0)]
    is_start = jnp.logical_or(i == 0, r != prev)

    @pl.when(jnp.logical_and(is_start, i > 0))
    def _ahead():
        issue(r + 1)

    @pl.when(is_start)
    def _wait():
        e = rexp_ref[0, r]
        pltpu.make_async_copy(w1_ref.at[e], b1.at[slot], sems.at[slot]).wait()
        pltpu.make_async_copy(w2_ref.at[e], b2.at[slot], sems.at[slot]).wait()
        pltpu.make_async_copy(w3_ref.at[e], b3.at[slot], sems.at[slot]).wait()

    xb = xg_ref[...]
    h1 = jax.lax.dot_general(xb, b1[slot], (((1,), (1,)), ((), ())),
                             preferred_element_type=jnp.float32)
    h3 = jax.lax.dot_general(xb, b3[slot], (((1,), (1,)), ((), ())),
                             preferred_element_type=jnp.float32)
    h = h1 * jax.nn.sigmoid(h1) * h3
    yg_ref[...] = jax.lax.dot_general(h, b2[slot], (((1,), (1,)), ((), ())),
                                      preferred_element_type=jnp.float32)


def _dispatch_body(x_hbm, posr_hbm, xg_hbm, idx_v, buf_v, sem):
    w = lax.axis_index("s") * NC + lax.axis_index("c")
    tok0 = (w % NS) * AB
    pltpu.sync_copy(posr_hbm.at[w], idx_v)
    pltpu.sync_copy(x_hbm.at[pl.ds(tok0, AB)], buf_v)
    pltpu.async_copy(buf_v, xg_hbm.at[idx_v], sem).wait()


def _dispatch(x2, posr):
    f = functools.partial(
        pl.kernel,
        out_type=jax.ShapeDtypeStruct((NP, D), jnp.float32),
        mesh=plsc.VectorSubcoreMesh(core_axis_name="c", subcore_axis_name="s"),
        scratch_types=[
            pltpu.VMEM((AB,), jnp.int32),
            pltpu.VMEM((AB, D), jnp.float32),
            pltpu.SemaphoreType.DMA,
        ],
    )(_dispatch_body)
    return f(x2, posr)


def _combine_body(yg_hbm, pos0_hbm, pos1_hbm, p0_hbm, p1_hbm, out_hbm,
                  idx0_v, idx1_v, pv0_v, pv1_v, rows0_v, rows1_v, sem):
    w = lax.axis_index("s") * NC + lax.axis_index("c")
    base = w * TB
    pltpu.sync_copy(pos0_hbm.at[pl.ds(base, TB)], idx0_v)
    pltpu.sync_copy(pos1_hbm.at[pl.ds(base, TB)], idx1_v)
    pltpu.sync_copy(p0_hbm.at[pl.ds(base, TB)], pv0_v)
    pltpu.sync_copy(p1_hbm.at[pl.ds(base, TB)], pv1_v)
    cp0 = pltpu.async_copy(yg_hbm.at[idx0_v], rows0_v, sem)
    cp1 = pltpu.async_copy(yg_hbm.at[idx1_v], rows1_v, sem)
    cp0.wait()
    cp1.wait()

    def grp(g, _):
        pv0 = pv0_v[pl.ds(g * 16, 16)]
        pv1 = pv1_v[pl.ds(g * 16, 16)]
        for i in range(16):
            s0 = pv0[i]
            s1 = pv1[i]
            r = g * 16 + i
            for j in range(D // 16):
                sl = pl.ds(j * 16, 16)
                rows0_v[r, sl] = s0 * rows0_v[r, sl] + s1 * rows1_v[r, sl]
        return _

    lax.fori_loop(0, TB // 16, grp, 0)
    pltpu.sync_copy(rows0_v, out_hbm.at[pl.ds(base, TB)])


def _combine(yg, pos0, pos1, p0, p1):
    f = functools.partial(
        pl.kernel,
        out_type=jax.ShapeDtypeStruct((N, D), jnp.float32),
        mesh=plsc.VectorSubcoreMesh(core_axis_name="c", subcore_axis_name="s"),
        scratch_types=[
            pltpu.VMEM((TB,), jnp.int32),
            pltpu.VMEM((TB,), jnp.int32),
            pltpu.VMEM((TB,), jnp.float32),
            pltpu.VMEM((TB,), jnp.float32),
            pltpu.VMEM((TB, D), jnp.float32),
            pltpu.VMEM((TB, D), jnp.float32),
            pltpu.SemaphoreType.DMA,
        ],
    )(_combine_body)
    return f(yg, pos0, pos1, p0, p1)


def kernel(x, gate_w, w1, w2, w3):
    x2 = x.reshape(N, D)

    pos0, pos1, p0, p1, rid, rexp, aux = pl.pallas_call(
        _router_kernel,
        in_specs=[
            pl.BlockSpec((N, D), lambda: (0, 0)),
            pl.BlockSpec((E, D), lambda: (0, 0)),
        ],
        out_specs=[
            pl.BlockSpec((1, N), lambda: (0, 0)),
            pl.BlockSpec((1, N), lambda: (0, 0)),
            pl.BlockSpec((1, N), lambda: (0, 0)),
            pl.BlockSpec((1, N), lambda: (0, 0)),
            pl.BlockSpec((1, NTILES), lambda: (0, 0)),
            pl.BlockSpec((1, E), lambda: (0, 0)),
            pl.BlockSpec((1, 1), lambda: (0, 0)),
        ],
        out_shape=[
            jax.ShapeDtypeStruct((1, N), jnp.int32),
            jax.ShapeDtypeStruct((1, N), jnp.int32),
            jax.ShapeDtypeStruct((1, N), jnp.float32),
            jax.ShapeDtypeStruct((1, N), jnp.float32),
            jax.ShapeDtypeStruct((1, NTILES), jnp.int32),
            jax.ShapeDtypeStruct((1, E), jnp.int32),
            jax.ShapeDtypeStruct((1, 1), jnp.float32),
        ],
    )(x2, gate_w)

    pos0 = pos0.reshape(N)
    pos1 = pos1.reshape(N)

    posr = jnp.concatenate([pos0, pos1]).reshape(NW, AB)
    xg = _dispatch(x2, posr)

    yg = pl.pallas_call(
        _group_kernel,
        grid_spec=pltpu.PrefetchScalarGridSpec(
            num_scalar_prefetch=2,
            grid=(NTILES,),
            in_specs=[
                pl.BlockSpec((TN, D), lambda i, rid, rexp: (i, 0)),
                pl.BlockSpec(memory_space=pl.ANY),
                pl.BlockSpec(memory_space=pl.ANY),
                pl.BlockSpec(memory_space=pl.ANY),
            ],
            out_specs=pl.BlockSpec((TN, D), lambda i, rid, rexp: (i, 0)),
            scratch_shapes=[
                pltpu.VMEM((2, H, D), jnp.float32),
                pltpu.VMEM((2, D, H), jnp.float32),
                pltpu.VMEM((2, H, D), jnp.float32),
                pltpu.SemaphoreType.DMA((2,)),
            ],
        ),
        out_shape=jax.ShapeDtypeStruct((NP, D), jnp.float32),
    )(rid, rexp, xg, w1, w2, w3)

    out = _combine(yg, pos0, pos1, p0.reshape(N), p1.reshape(N))
    return out.reshape(x.shape), aux.reshape(())


# speedup vs baseline: 1.3132x; 1.3132x over previous
import functools

import jax
from jax import lax
import jax.numpy as jnp
from jax.experimental import pallas as pl
from jax.experimental.pallas import tpu as pltpu
from jax.experimental.pallas import tpu_sc as plsc

N, D, E, H, TOPK = 2048, 768, 8, 2048, 2
AUX_COEF = 0.01
TN = 256
NTILES = 24
NP = NTILES * TN

NC, NS = 2, 16
NW = NC * NS
AB = (2 * N) // NW
TB = N // NW


def _router_kernel(x_ref, gate_ref, pos0_ref, pos1_ref, p0_ref, p1_ref,
                   rid_ref, rexp_ref, aux_ref):
    lt = jax.lax.dot_general(gate_ref[...], x_ref[...],
                             (((1,), (1,)), ((), ())),
                             preferred_element_type=jnp.float32)
    m = jnp.max(lt, axis=0, keepdims=True)
    ex = jnp.exp(lt - m)
    p = ex / jnp.sum(ex, axis=0, keepdims=True)

    tpe = jnp.mean(p, axis=1, keepdims=True)
    aux = AUX_COEF * jnp.mean((tpe - 1.0 / E) ** 2)
    aux_ref[...] = aux.reshape(1, 1)

    ie = jax.lax.broadcasted_iota(jnp.int32, p.shape, 0)
    m1 = jnp.max(p, axis=0, keepdims=True)
    i1 = jnp.min(jnp.where(p == m1, ie, E), axis=0, keepdims=True)
    f1 = ie == i1
    pw = jnp.where(f1, -jnp.inf, p)
    m2 = jnp.max(pw, axis=0, keepdims=True)
    i2 = jnp.min(jnp.where(pw == m2, ie, E), axis=0, keepdims=True)
    f2 = ie == i2
    denom = m1 + m2
    p0_ref[...] = m1 / denom
    p1_ref[...] = m2 / denom

    oh0 = f1.astype(jnp.float32)
    oh1 = f2.astype(jnp.float32)

    it_r = jax.lax.broadcasted_iota(jnp.int32, (N, N), 0)
    it_c = jax.lax.broadcasted_iota(jnp.int32, (N, N), 1)
    tri = (it_r < it_c).astype(jnp.float32)
    rank0t = jax.lax.dot_general(oh0, tri, (((1,), (0,)), ((), ())),
                                 preferred_element_type=jnp.float32)
    rank1t = jax.lax.dot_general(oh1, tri, (((1,), (0,)), ((), ())),
                                 preferred_element_type=jnp.float32)
    rank0 = jnp.sum(rank0t * oh0, axis=0, keepdims=True)
    rank1 = jnp.sum(rank1t * oh1, axis=0, keepdims=True)

    c0 = jnp.sum(oh0, axis=1, keepdims=True)
    c1 = jnp.sum(oh1, axis=1, keepdims=True)
    ci = (c0 + c1).astype(jnp.int32)
    pc = jnp.right_shift(ci + (TN - 1), 8) << 8
    ie8r = jax.lax.broadcasted_iota(jnp.int32, (E, E), 0)
    ie8c = jax.lax.broadcasted_iota(jnp.int32, (E, E), 1)
    tri8 = (ie8r > ie8c).astype(jnp.float32)
    po = jax.lax.dot_general(tri8, pc.astype(jnp.float32),
                             (((1,), (0,)), ((), ())),
                             preferred_element_type=jnp.float32)

    pos0 = jnp.sum(po * oh0, axis=0, keepdims=True) + rank0
    pos1 = jnp.sum((po + c0) * oh1, axis=0, keepdims=True) + rank1
    pos0_ref[...] = pos0.astype(jnp.int32)
    pos1_ref[...] = pos1.astype(jnp.int32)

    nz = (pc > 0).astype(jnp.float32)
    itile = (jax.lax.broadcasted_iota(jnp.int32, (1, NTILES), 1)
             .astype(jnp.float32) * float(TN))
    started = nz * (po <= itile).astype(jnp.float32)
    rid_ref[...] = jnp.sum(started, axis=0, keepdims=True).astype(jnp.int32) - 1
    tri8i = (ie8r >= ie8c).astype(jnp.float32)
    nzcum = jax.lax.dot_general(tri8i, nz, (((1,), (0,)), ((), ())),
                                preferred_element_type=jnp.float32)
    rank_e = nzcum - 1.0
    ir = (jax.lax.broadcasted_iota(jnp.int32, (E, E), 1)
          .astype(jnp.float32))
    ev = (jax.lax.broadcasted_iota(jnp.int32, (E, E), 0)
          .astype(jnp.float32))
    msk = nz * (rank_e == ir).astype(jnp.float32)
    rexp = jnp.sum((ev + 1.0) * msk, axis=0, keepdims=True) - 1.0
    rexp_ref[...] = rexp.astype(jnp.int32)


def _group_kernel(rid_ref, rexp_ref, xg_ref, w1_ref, w2_ref, w3_ref, yg_ref,
                  b1, b2, b3, semA, semB):
    i = pl.program_id(0)
    r = rid_ref[0, i]
    sl3 = lax.rem(r, 3)
    sl2 = lax.rem(r, 2)

    def issue13(rr):
        sl = lax.rem(rr, 3)
        e = rexp_ref[0, jnp.minimum(rr, E - 1)]

        @pl.when(jnp.logical_and(rr <= E - 1, e >= 0))
        def _():
            pltpu.make_async_copy(w1_ref.at[e], b1.at[sl], semA.at[sl]).start()
            pltpu.make_async_copy(w3_ref.at[e], b3.at[sl], semA.at[sl]).start()

    def issue2(rr):
        sl = lax.rem(rr, 2)
        e = rexp_ref[0, jnp.minimum(rr, E - 1)]

        @pl.when(jnp.logical_and(rr <= E - 1, e >= 0))
        def _():
            pltpu.make_async_copy(w2_ref.at[e], b2.at[sl], semB.at[sl]).start()

    @pl.when(i == 0)
    def _prime():
        issue13(jnp.int32(0))
        issue13(jnp.int32(1))
        issue13(jnp.int32(2))
        issue2(jnp.int32(0))
        issue2(jnp.int32(1))

    prev = rid_ref[0, jnp.maximum(i - 1, 0)]
    is_start = jnp.logical_or(i == 0, r != prev)

    @pl.when(jnp.logical_and(is_start, i > 0))
    def _ahead():
        issue13(r + 2)
        issue2(r + 1)

    @pl.when(is_start)
    def _wait():
        e = rexp_ref[0, r]
        pltpu.make_async_copy(w1_ref.at[e], b1.at[sl3], semA.at[sl3]).wait()
        pltpu.make_async_copy(w3_ref.at[e], b3.at[sl3], semA.at[sl3]).wait()
        pltpu.make_async_copy(w2_ref.at[e], b2.at[sl2], semB.at[sl2]).wait()

    xb = xg_ref[...]
    h1 = jax.lax.dot_general(xb, b1[sl3], (((1,), (1,)), ((), ())),
                             preferred_element_type=jnp.float32)
    h3 = jax.lax.dot_general(xb, b3[sl3], (((1,), (1,)), ((), ())),
                             preferred_element_type=jnp.float32)
    h = h1 * jax.nn.sigmoid(h1) * h3
    yg_ref[...] = jax.lax.dot_general(h, b2[sl2], (((1,), (1,)), ((), ())),
                                      preferred_element_type=jnp.float32)


def _dispatch_body(x_hbm, posr_hbm, xg_hbm, idx_v, buf_v, sem):
    w = lax.axis_index("s") * NC + lax.axis_index("c")
    tok0 = (w % NS) * AB
    pltpu.sync_copy(posr_hbm.at[w], idx_v)
    pltpu.sync_copy(x_hbm.at[pl.ds(tok0, AB)], buf_v)
    pltpu.async_copy(buf_v, xg_hbm.at[idx_v], sem).wait()


def _dispatch(x2, posr):
    f = functools.partial(
        pl.kernel,
        out_type=jax.ShapeDtypeStruct((NP, D), jnp.float32),
        mesh=plsc.VectorSubcoreMesh(core_axis_name="c", subcore_axis_name="s"),
        scratch_types=[
            pltpu.VMEM((AB,), jnp.int32),
            pltpu.VMEM((AB, D), jnp.float32),
            pltpu.SemaphoreType.DMA,
        ],
    )(_dispatch_body)
    return f(x2, posr)


def _combine_body(yg_hbm, pos0_hbm, pos1_hbm, p0_hbm, p1_hbm, out_hbm,
                  idx0_v, idx1_v, pv0_v, pv1_v, rows0_v, rows1_v, sem):
    w = lax.axis_index("s") * NC + lax.axis_index("c")
    base = w * TB
    pltpu.sync_copy(pos0_hbm.at[pl.ds(base, TB)], idx0_v)
    pltpu.sync_copy(pos1_hbm.at[pl.ds(base, TB)], idx1_v)
    pltpu.sync_copy(p0_hbm.at[pl.ds(base, TB)], pv0_v)
    pltpu.sync_copy(p1_hbm.at[pl.ds(base, TB)], pv1_v)
    cp0 = pltpu.async_copy(yg_hbm.at[idx0_v], rows0_v, sem)
    cp1 = pltpu.async_copy(yg_hbm.at[idx1_v], rows1_v, sem)
    cp0.wait()
    cp1.wait()

    def grp(g, _):
        pv0 = pv0_v[pl.ds(g * 16, 16)]
        pv1 = pv1_v[pl.ds(g * 16, 16)]
        for i in range(16):
            s0 = pv0[i]
            s1 = pv1[i]
            r = g * 16 + i
            for j in range(D // 16):
                sl = pl.ds(j * 16, 16)
                rows0_v[r, sl] = s0 * rows0_v[r, sl] + s1 * rows1_v[r, sl]
        return _

    lax.fori_loop(0, TB // 16, grp, 0)
    pltpu.sync_copy(rows0_v, out_hbm.at[pl.ds(base, TB)])


def _combine(yg, pos0, pos1, p0, p1):
    f = functools.partial(
        pl.kernel,
        out_type=jax.ShapeDtypeStruct((N, D), jnp.float32),
        mesh=plsc.VectorSubcoreMesh(core_axis_name="c", subcore_axis_name="s"),
        scratch_types=[
            pltpu.VMEM((TB,), jnp.int32),
            pltpu.VMEM((TB,), jnp.int32),
            pltpu.VMEM((TB,), jnp.float32),
            pltpu.VMEM((TB,), jnp.float32),
            pltpu.VMEM((TB, D), jnp.float32),
            pltpu.VMEM((TB, D), jnp.float32),
            pltpu.SemaphoreType.DMA,
        ],
    )(_combine_body)
    return f(yg, pos0, pos1, p0, p1)


def kernel(x, gate_w, w1, w2, w3):
    x2 = x.reshape(N, D)

    pos0, pos1, p0, p1, rid, rexp, aux = pl.pallas_call(
        _router_kernel,
        in_specs=[
            pl.BlockSpec((N, D), lambda: (0, 0)),
            pl.BlockSpec((E, D), lambda: (0, 0)),
        ],
        out_specs=[
            pl.BlockSpec((1, N), lambda: (0, 0)),
            pl.BlockSpec((1, N), lambda: (0, 0)),
            pl.BlockSpec((1, N), lambda: (0, 0)),
            pl.BlockSpec((1, N), lambda: (0, 0)),
            pl.BlockSpec((1, NTILES), lambda: (0, 0)),
            pl.BlockSpec((1, E), lambda: (0, 0)),
            pl.BlockSpec((1, 1), lambda: (0, 0)),
        ],
        out_shape=[
            jax.ShapeDtypeStruct((1, N), jnp.int32),
            jax.ShapeDtypeStruct((1, N), jnp.int32),
            jax.ShapeDtypeStruct((1, N), jnp.float32),
            jax.ShapeDtypeStruct((1, N), jnp.float32),
            jax.ShapeDtypeStruct((1, NTILES), jnp.int32),
            jax.ShapeDtypeStruct((1, E), jnp.int32),
            jax.ShapeDtypeStruct((1, 1), jnp.float32),
        ],
    )(x2, gate_w)

    pos0 = pos0.reshape(N)
    pos1 = pos1.reshape(N)

    posr = jnp.concatenate([pos0, pos1]).reshape(NW, AB)
    xg = _dispatch(x2, posr)

    yg = pl.pallas_call(
        _group_kernel,
        grid_spec=pltpu.PrefetchScalarGridSpec(
            num_scalar_prefetch=2,
            grid=(NTILES,),
            in_specs=[
                pl.BlockSpec((TN, D), lambda i, rid, rexp: (i, 0)),
                pl.BlockSpec(memory_space=pl.ANY),
                pl.BlockSpec(memory_space=pl.ANY),
                pl.BlockSpec(memory_space=pl.ANY),
            ],
            out_specs=pl.BlockSpec((TN, D), lambda i, rid, rexp: (i, 0)),
            scratch_shapes=[
                pltpu.VMEM((3, H, D), jnp.float32),
                pltpu.VMEM((2, D, H), jnp.float32),
                pltpu.VMEM((3, H, D), jnp.float32),
                pltpu.SemaphoreType.DMA((3,)),
                pltpu.SemaphoreType.DMA((2,)),
            ],
        ),
        out_shape=jax.ShapeDtypeStruct((NP, D), jnp.float32),
    )(rid, rexp, xg, w1, w2, w3)

    out = _combine(yg, pos0, pos1, p0.reshape(N), p1.reshape(N))
    return out.reshape(x.shape), aux.reshape(())


# speedup vs baseline: 1.3692x; 1.0426x over previous
import functools

import jax
from jax import lax
import jax.numpy as jnp
from jax.experimental import pallas as pl
from jax.experimental.pallas import tpu as pltpu
from jax.experimental.pallas import tpu_sc as plsc

N, D, E, H, TOPK = 2048, 768, 8, 2048, 2
AUX_COEF = 0.01
TN = 256
NTILES = 24
NP = NTILES * TN

NC, NS = 2, 16
NW = NC * NS
AB = (2 * N) // NW
TB = N // NW


def _router_kernel(x_ref, gate_ref, pos0_ref, pos1_ref, p0_ref, p1_ref,
                   rid_ref, rexp_ref, aux_ref):
    lt = jax.lax.dot_general(gate_ref[...], x_ref[...],
                             (((1,), (1,)), ((), ())),
                             preferred_element_type=jnp.float32)
    m = jnp.max(lt, axis=0, keepdims=True)
    ex = jnp.exp(lt - m)
    p = ex / jnp.sum(ex, axis=0, keepdims=True)

    tpe = jnp.mean(p, axis=1, keepdims=True)
    aux = AUX_COEF * jnp.mean((tpe - 1.0 / E) ** 2)
    aux_ref[...] = aux.reshape(1, 1)

    ie = jax.lax.broadcasted_iota(jnp.int32, p.shape, 0)
    m1 = jnp.max(p, axis=0, keepdims=True)
    i1 = jnp.min(jnp.where(p == m1, ie, E), axis=0, keepdims=True)
    f1 = ie == i1
    pw = jnp.where(f1, -jnp.inf, p)
    m2 = jnp.max(pw, axis=0, keepdims=True)
    i2 = jnp.min(jnp.where(pw == m2, ie, E), axis=0, keepdims=True)
    f2 = ie == i2
    denom = m1 + m2
    p0_ref[...] = m1 / denom
    p1_ref[...] = m2 / denom

    oh0 = f1.astype(jnp.float32)
    oh1 = f2.astype(jnp.float32)

    it_r = jax.lax.broadcasted_iota(jnp.int32, (N, N), 0)
    it_c = jax.lax.broadcasted_iota(jnp.int32, (N, N), 1)
    tri = (it_r < it_c).astype(jnp.float32)
    rank0t = jax.lax.dot_general(oh0, tri, (((1,), (0,)), ((), ())),
                                 preferred_element_type=jnp.float32)
    rank1t = jax.lax.dot_general(oh1, tri, (((1,), (0,)), ((), ())),
                                 preferred_element_type=jnp.float32)
    rank0 = jnp.sum(rank0t * oh0, axis=0, keepdims=True)
    rank1 = jnp.sum(rank1t * oh1, axis=0, keepdims=True)

    c0 = jnp.sum(oh0, axis=1, keepdims=True)
    c1 = jnp.sum(oh1, axis=1, keepdims=True)
    ci = (c0 + c1).astype(jnp.int32)
    pc = jnp.right_shift(ci + (TN - 1), 8) << 8
    ie8r = jax.lax.broadcasted_iota(jnp.int32, (E, E), 0)
    ie8c = jax.lax.broadcasted_iota(jnp.int32, (E, E), 1)
    tri8 = (ie8r > ie8c).astype(jnp.float32)
    po = jax.lax.dot_general(tri8, pc.astype(jnp.float32),
                             (((1,), (0,)), ((), ())),
                             preferred_element_type=jnp.float32)

    pos0 = jnp.sum(po * oh0, axis=0, keepdims=True) + rank0
    pos1 = jnp.sum((po + c0) * oh1, axis=0, keepdims=True) + rank1
    pos0_ref[...] = pos0.astype(jnp.int32)
    pos1_ref[...] = pos1.astype(jnp.int32)

    nz = (pc > 0).astype(jnp.float32)
    itile = (jax.lax.broadcasted_iota(jnp.int32, (1, NTILES), 1)
             .astype(jnp.float32) * float(TN))
    started = nz * (po <= itile).astype(jnp.float32)
    rid_ref[...] = jnp.sum(started, axis=0, keepdims=True).astype(jnp.int32) - 1
    tri8i = (ie8r >= ie8c).astype(jnp.float32)
    nzcum = jax.lax.dot_general(tri8i, nz, (((1,), (0,)), ((), ())),
                                preferred_element_type=jnp.float32)
    rank_e = nzcum - 1.0
    ir = (jax.lax.broadcasted_iota(jnp.int32, (E, E), 1)
          .astype(jnp.float32))
    ev = (jax.lax.broadcasted_iota(jnp.int32, (E, E), 0)
          .astype(jnp.float32))
    msk = nz * (rank_e == ir).astype(jnp.float32)
    rexp = jnp.sum((ev + 1.0) * msk, axis=0, keepdims=True) - 1.0
    rexp_ref[...] = rexp.astype(jnp.int32)


def _group_kernel(rid_ref, rexp_ref, xg_ref, w1_ref, w2_ref, w3_ref, yg_ref,
                  b1, b2, b3, sems):
    i = pl.program_id(0)
    r = rid_ref[0, i]
    slot = lax.rem(r, 2)

    def issue(rr):
        sl = lax.rem(rr, 2)
        e = rexp_ref[0, jnp.minimum(rr, E - 1)]

        @pl.when(jnp.logical_and(rr <= E - 1, e >= 0))
        def _():
            for lo, sz in ((0, H // 2), (H // 2, H // 2)):
                pltpu.make_async_copy(w1_ref.at[e, pl.ds(lo, sz)],
                                      b1.at[sl, pl.ds(lo, sz)],
                                      sems.at[sl]).start()
                pltpu.make_async_copy(w3_ref.at[e, pl.ds(lo, sz)],
                                      b3.at[sl, pl.ds(lo, sz)],
                                      sems.at[sl]).start()
            for lo, sz in ((0, D // 2), (D // 2, D // 2)):
                pltpu.make_async_copy(w2_ref.at[e, pl.ds(lo, sz)],
                                      b2.at[sl, pl.ds(lo, sz)],
                                      sems.at[sl]).start()

    @pl.when(i == 0)
    def _prime():
        issue(jnp.int32(0))
        issue(jnp.int32(1))

    prev = rid_ref[0, jnp.maximum(i - 1, 0)]
    is_start = jnp.logical_or(i == 0, r != prev)

    @pl.when(jnp.logical_and(is_start, i > 0))
    def _ahead():
        issue(r + 1)

    @pl.when(is_start)
    def _wait():
        e = rexp_ref[0, r]
        for lo, sz in ((0, H // 2), (H // 2, H // 2)):
            pltpu.make_async_copy(w1_ref.at[e, pl.ds(lo, sz)],
                                  b1.at[slot, pl.ds(lo, sz)],
                                  sems.at[slot]).wait()
            pltpu.make_async_copy(w3_ref.at[e, pl.ds(lo, sz)],
                                  b3.at[slot, pl.ds(lo, sz)],
                                  sems.at[slot]).wait()
        for lo, sz in ((0, D // 2), (D // 2, D // 2)):
            pltpu.make_async_copy(w2_ref.at[e, pl.ds(lo, sz)],
                                  b2.at[slot, pl.ds(lo, sz)],
                                  sems.at[slot]).wait()

    xb = xg_ref[...]
    h1 = jax.lax.dot_general(xb, b1[slot], (((1,), (1,)), ((), ())),
                             preferred_element_type=jnp.float32)
    h3 = jax.lax.dot_general(xb, b3[slot], (((1,), (1,)), ((), ())),
                             preferred_element_type=jnp.float32)
    h = h1 * jax.nn.sigmoid(h1) * h3
    yg_ref[...] = jax.lax.dot_general(h, b2[slot], (((1,), (1,)), ((), ())),
                                      preferred_element_type=jnp.float32)


def _dispatch_body(x_hbm, posr_hbm, xg_hbm, idx_v, buf_v, sem):
    w = lax.axis_index("s") * NC + lax.axis_index("c")
    tok0 = (w % NS) * AB
    pltpu.sync_copy(posr_hbm.at[w], idx_v)
    pltpu.sync_copy(x_hbm.at[pl.ds(tok0, AB)], buf_v)
    pltpu.async_copy(buf_v, xg_hbm.at[idx_v], sem).wait()


def _dispatch(x2, posr):
    f = functools.partial(
        pl.kernel,
        out_type=jax.ShapeDtypeStruct((NP, D), jnp.float32),
        mesh=plsc.VectorSubcoreMesh(core_axis_name="c", subcore_axis_name="s"),
        scratch_types=[
            pltpu.VMEM((AB,), jnp.int32),
            pltpu.VMEM((AB, D), jnp.float32),
            pltpu.SemaphoreType.DMA,
        ],
    )(_dispatch_body)
    return f(x2, posr)


def _combine_body(yg_hbm, pos0_hbm, pos1_hbm, p0_hbm, p1_hbm, out_hbm,
                  idx0_v, idx1_v, pv0_v, pv1_v, rows0_v, rows1_v, sem):
    w = lax.axis_index("s") * NC + lax.axis_index("c")
    base = w * TB
    pltpu.sync_copy(pos0_hbm.at[pl.ds(base, TB)], idx0_v)
    pltpu.sync_copy(pos1_hbm.at[pl.ds(base, TB)], idx1_v)
    pltpu.sync_copy(p0_hbm.at[pl.ds(base, TB)], pv0_v)
    pltpu.sync_copy(p1_hbm.at[pl.ds(base, TB)], pv1_v)
    cp0 = pltpu.async_copy(yg_hbm.at[idx0_v], rows0_v, sem)
    cp1 = pltpu.async_copy(yg_hbm.at[idx1_v], rows1_v, sem)
    cp0.wait()
    cp1.wait()

    def grp(g, _):
        pv0 = pv0_v[pl.ds(g * 16, 16)]
        pv1 = pv1_v[pl.ds(g * 16, 16)]
        for i in range(16):
            s0 = pv0[i]
            s1 = pv1[i]
            r = g * 16 + i
            for j in range(D // 16):
                sl = pl.ds(j * 16, 16)
                rows0_v[r, sl] = s0 * rows0_v[r, sl] + s1 * rows1_v[r, sl]
        return _

    lax.fori_loop(0, TB // 16, grp, 0)
    pltpu.sync_copy(rows0_v, out_hbm.at[pl.ds(base, TB)])


def _combine(yg, pos0, pos1, p0, p1):
    f = functools.partial(
        pl.kernel,
        out_type=jax.ShapeDtypeStruct((N, D), jnp.float32),
        mesh=plsc.VectorSubcoreMesh(core_axis_name="c", subcore_axis_name="s"),
        scratch_types=[
            pltpu.VMEM((TB,), jnp.int32),
            pltpu.VMEM((TB,), jnp.int32),
            pltpu.VMEM((TB,), jnp.float32),
            pltpu.VMEM((TB,), jnp.float32),
            pltpu.VMEM((TB, D), jnp.float32),
            pltpu.VMEM((TB, D), jnp.float32),
            pltpu.SemaphoreType.DMA,
        ],
    )(_combine_body)
    return f(yg, pos0, pos1, p0, p1)


def kernel(x, gate_w, w1, w2, w3):
    x2 = x.reshape(N, D)

    pos0, pos1, p0, p1, rid, rexp, aux = pl.pallas_call(
        _router_kernel,
        in_specs=[
            pl.BlockSpec((N, D), lambda: (0, 0)),
            pl.BlockSpec((E, D), lambda: (0, 0)),
        ],
        out_specs=[
            pl.BlockSpec((1, N), lambda: (0, 0)),
            pl.BlockSpec((1, N), lambda: (0, 0)),
            pl.BlockSpec((1, N), lambda: (0, 0)),
            pl.BlockSpec((1, N), lambda: (0, 0)),
            pl.BlockSpec((1, NTILES), lambda: (0, 0)),
            pl.BlockSpec((1, E), lambda: (0, 0)),
            pl.BlockSpec((1, 1), lambda: (0, 0)),
        ],
        out_shape=[
            jax.ShapeDtypeStruct((1, N), jnp.int32),
            jax.ShapeDtypeStruct((1, N), jnp.int32),
            jax.ShapeDtypeStruct((1, N), jnp.float32),
            jax.ShapeDtypeStruct((1, N), jnp.float32),
            jax.ShapeDtypeStruct((1, NTILES), jnp.int32),
            jax.ShapeDtypeStruct((1, E), jnp.int32),
            jax.ShapeDtypeStruct((1, 1), jnp.float32),
        ],
    )(x2, gate_w)

    pos0 = pos0.reshape(N)
    pos1 = pos1.reshape(N)

    posr = jnp.concatenate([pos0, pos1]).reshape(NW, AB)
    xg = _dispatch(x2, posr)

    yg = pl.pallas_call(
        _group_kernel,
        grid_spec=pltpu.PrefetchScalarGridSpec(
            num_scalar_prefetch=2,
            grid=(NTILES,),
            in_specs=[
                pl.BlockSpec((TN, D), lambda i, rid, rexp: (i, 0)),
                pl.BlockSpec(memory_space=pl.ANY),
                pl.BlockSpec(memory_space=pl.ANY),
                pl.BlockSpec(memory_space=pl.ANY),
            ],
            out_specs=pl.BlockSpec((TN, D), lambda i, rid, rexp: (i, 0)),
            scratch_shapes=[
                pltpu.VMEM((2, H, D), jnp.float32),
                pltpu.VMEM((2, D, H), jnp.float32),
                pltpu.VMEM((2, H, D), jnp.float32),
                pltpu.SemaphoreType.DMA((2,)),
            ],
        ),
        out_shape=jax.ShapeDtypeStruct((NP, D), jnp.float32),
    )(rid, rexp, xg, w1, w2, w3)

    out = _combine(yg, pos0, pos1, p0.reshape(N), p1.reshape(N))
    return out.reshape(x.shape), aux.reshape(())
